# baseline (speedup 1.0000x reference)
import functools

import jax
import jax.numpy as jnp
from jax import lax
from jax.experimental import pallas as pl
from jax.experimental.pallas import tpu as pltpu
from jax.experimental.pallas import tpu_sc as plsc

_HI = jax.lax.Precision.HIGHEST

_NW = 32
_RPW = 32
_SC_ROWS = _NW * _RPW
_CH = 8


def _lane_perm(v, idx):
    return lax.gather(
        v, idx[:, None],
        lax.GatherDimensionNumbers(offset_dims=(), collapsed_slice_dims=(0,),
                                   start_index_map=(0,)),
        slice_sizes=(1,), mode=lax.GatherScatterMode.PROMISE_IN_BOUNDS)


def _lane_allsum(v):
    lanes = jnp.arange(16, dtype=jnp.int32)
    for stride in (1, 2, 4, 8):
        v = v + _lane_perm(v, lanes ^ stride)
    return v


def _sc_qk_kernel(wq_hbm, wk_hbm, bq_hbm, out_hbm, wq_v, wk_v, bq_v, qk_v,
                  *, d, base0):
    w = lax.axis_index("s") * 2 + lax.axis_index("c")
    base = base0 + w * _RPW
    nv = d // 16

    def _zero(k, _):
        qk_v[pl.ds(k * 16, 16)] = jnp.zeros((16,), jnp.float32)
        return 0

    lax.fori_loop(0, nv, _zero, 0)
    pltpu.sync_copy(bq_hbm.at[pl.ds(base, _RPW)], bq_v.at[pl.ds(0, _RPW)])

    def _chunk(c, _):
        row0 = base + c * _CH
        pltpu.sync_copy(wq_hbm.at[pl.ds(row0, _CH)], wq_v)
        pltpu.sync_copy(wk_hbm.at[pl.ds(row0, _CH)], wk_v)
        bvec = bq_v[pl.ds(c * _CH, 16)]
        for r in range(_CH):
            def _rowsum(k, acc):
                return acc + wq_v[r, pl.ds(k * 16, 16)]

            acc = lax.fori_loop(0, nv, _rowsum, jnp.zeros((16,), jnp.float32))
            qr = _lane_allsum(acc) + bvec[r]

            def _accum(k, _):
                sl = pl.ds(k * 16, 16)
                qk_v[sl] = qk_v[sl] + qr * wk_v[r, sl]
                return 0

            lax.fori_loop(0, nv, _accum, 0)
        return 0

    lax.fori_loop(0, _RPW // _CH, _chunk, 0)
    pltpu.sync_copy(qk_v, out_hbm.at[w])


def _qk_tc_kernel(wq_ref, bq_ref, wk_ref, qk_ref):
    i = pl.program_id(0)
    qc = jnp.sum(wq_ref[...], axis=1) + bq_ref[0, :]
    part = jnp.sum(qc[:, None] * wk_ref[...], axis=0, keepdims=True)

    @pl.when(i == 0)
    def _():
        qk_ref[...] = jnp.zeros_like(qk_ref)

    qk_ref[...] += part


def _tail_kernel(qk_in_ref, h_ref, wv_ref, bv_ref, we_ref, be_ref,
                 expert_ref, pmax_ref, logits_ref,
                 hbar_ref, acc_ref, m_ref, s_ref,
                 *, s2, s3, inv_scale):
    i = pl.program_id(0)

    @pl.when(i < s2)
    def _phase2():
        @pl.when(i == 0)
        def _():
            m_ref[...] = jnp.full_like(m_ref, -jnp.inf)
            s_ref[...] = jnp.zeros_like(s_ref)
            acc_ref[...] = jnp.zeros_like(acc_ref)

        h = h_ref[...]
        t = jax.lax.dot_general(
            h, qk_in_ref[0, :], (((2,), (0,)), ((), ())),
            preferred_element_type=jnp.float32, precision=_HI)
        t = t * inv_scale
        m_prev = m_ref[...]
        m_new = jnp.maximum(m_prev, jnp.max(t, axis=1, keepdims=True))
        alpha = jnp.exp(m_prev - m_new)
        p = jnp.exp(t - m_new)
        s_ref[...] = s_ref[...] * alpha + jnp.sum(p, axis=1, keepdims=True)
        pv = jax.lax.dot_general(
            p, h, (((1,), (1,)), ((0,), (0,))),
            preferred_element_type=jnp.float32)
        acc_ref[...] = acc_ref[...] * alpha + pv
        m_ref[...] = m_new

        @pl.when(i == s2 - 1)
        def _():
            hbar_ref[...] = acc_ref[...] / s_ref[...]

    @pl.when(i >= s2)
    def _phase3():
        hb = hbar_ref[...]
        hb_hi = hb.astype(jnp.bfloat16)
        hb_lo = (hb - hb_hi.astype(jnp.float32)).astype(jnp.bfloat16)
        hb2 = jnp.concatenate([hb_hi, hb_lo], axis=0)
        wv_hi = wv_ref[...].astype(jnp.bfloat16)
        bdim = hb.shape[0]
        rr = jax.lax.dot_general(
            hb2, wv_hi, (((1,), (1,)), ((), ())),
            preferred_element_type=jnp.float32)
        r = rr[:bdim, :] + rr[bdim:, :] + bv_ref[...]
        part = jax.lax.dot_general(
            r, we_ref[...], (((1,), (1,)), ((), ())),
            preferred_element_type=jnp.float32, precision=_HI)

        @pl.when(i == s2)
        def _():
            logits_ref[...] = jnp.zeros_like(logits_ref)

        logits_ref[...] += part

        @pl.when(i == s2 + s3 - 1)
        def _():
            logits = logits_ref[...] + be_ref[...]
            logits_ref[...] = logits
            row_max = jnp.max(logits, axis=1, keepdims=True)
            ex = jnp.exp(logits - row_max)
            denom = jnp.sum(ex, axis=1, keepdims=True)
            pmax_ref[...] = jnp.max(ex, axis=1, keepdims=True) / denom
            bd, ed = logits.shape
            idx = jax.lax.broadcasted_iota(jnp.int32, (bd, ed), 1)
            am = jnp.min(jnp.where(logits == row_max, idx, ed),
                         axis=1, keepdims=True)
            expert_ref[...] = (idx == am).astype(jnp.int32)


def kernel(h_dense, Wq, bq, Wk, bk, Wv, bv, We, be):
    del bk
    B, N, D = h_dense.shape
    E = We.shape[0]
    f32 = jnp.float32
    tc_rows = D - _SC_ROWS

    sc_fn = functools.partial(_sc_qk_kernel, d=D, base0=tc_rows)
    qk_sc_rows = pl.kernel(
        sc_fn,
        out_type=jax.ShapeDtypeStruct((_NW, D), f32),
        mesh=plsc.VectorSubcoreMesh(core_axis_name="c", subcore_axis_name="s"),
        scratch_types=[
            pltpu.VMEM((_CH, D), f32),
            pltpu.VMEM((_CH, D), f32),
            pltpu.VMEM((_RPW + 16,), f32),
            pltpu.VMEM((D,), f32),
        ],
    )(Wq, Wk, bq)

    C1 = 512
    s1 = tc_rows // C1
    qk_tc = pl.pallas_call(
        _qk_tc_kernel,
        grid=(s1,),
        in_specs=[
            pl.BlockSpec((C1, D), lambda i: (i, 0)),
            pl.BlockSpec((1, C1), lambda i: (0, i)),
            pl.BlockSpec((C1, D), lambda i: (i, 0)),
        ],
        out_specs=pl.BlockSpec((1, D), lambda i: (0, 0)),
        out_shape=jax.ShapeDtypeStruct((1, D), f32),
    )(Wq, bq.reshape(1, D), Wk)

    qk = qk_tc + jnp.sum(qk_sc_rows, axis=0, keepdims=True)

    C2, C3 = 256, 256
    s2, s3 = N // C2, D // C3
    l2, l3 = s2 - 1, s3 - 1

    def _clip(v, hi):
        return jnp.minimum(jnp.maximum(v, 0), hi)

    expert, pmax, logits = pl.pallas_call(
        functools.partial(_tail_kernel, s2=s2, s3=s3,
                          inv_scale=1.0 / (float(D) ** 0.5)),
        grid=(s2 + s3,),
        in_specs=[
            pl.BlockSpec((1, D), lambda i: (0, 0)),
            pl.BlockSpec((B, C2, D), lambda i: (0, _clip(i, l2), 0)),
            pl.BlockSpec((C3, D), lambda i: (_clip(i - s2, l3), 0)),
            pl.BlockSpec((1, C3), lambda i: (0, _clip(i - s2, l3))),
            pl.BlockSpec((E, C3), lambda i: (0, _clip(i - s2, l3))),
            pl.BlockSpec((1, E), lambda i: (0, 0)),
        ],
        out_specs=[
            pl.BlockSpec((B, E), lambda i: (0, 0)),
            pl.BlockSpec((B, 1), lambda i: (0, 0)),
            pl.BlockSpec((B, E), lambda i: (0, 0)),
        ],
        out_shape=[
            jax.ShapeDtypeStruct((B, E), jnp.int32),
            jax.ShapeDtypeStruct((B, 1), f32),
            jax.ShapeDtypeStruct((B, E), f32),
        ],
        scratch_shapes=[
            pltpu.VMEM((B, D), f32),
            pltpu.VMEM((B, D), f32),
            pltpu.VMEM((B, 1), f32),
            pltpu.VMEM((B, 1), f32),
        ],
    )(qk, h_dense, Wv, bv.reshape(1, D), We, be.reshape(1, E))

    return (expert, pmax, logits)


# speedup vs baseline: 1.1618x; 1.1618x over previous
import functools

import jax
import jax.numpy as jnp
from jax import lax
from jax.experimental import pallas as pl
from jax.experimental.pallas import tpu as pltpu
from jax.experimental.pallas import tpu_sc as plsc

_HI = jax.lax.Precision.HIGHEST

_NW = 32
_RPW = 64
_SC_ROWS = _NW * _RPW
_CH = 4


def _lane_perm(v, idx):
    return lax.gather(
        v, idx[:, None],
        lax.GatherDimensionNumbers(offset_dims=(), collapsed_slice_dims=(0,),
                                   start_index_map=(0,)),
        slice_sizes=(1,), mode=lax.GatherScatterMode.PROMISE_IN_BOUNDS)


def _lane_allsum(v):
    lanes = jnp.arange(16, dtype=jnp.int32)
    for stride in (1, 2, 4, 8):
        v = v + _lane_perm(v, lanes ^ stride)
    return v


def _sc_qk_kernel(wq_hbm, wk_hbm, bq_hbm, out_hbm,
                  wq_a, wq_b, wk_a, wk_b, bq_v, qk_v, sem_a, sem_b,
                  *, d, base0):
    w = lax.axis_index("s") * 2 + lax.axis_index("c")
    base = base0 + w * _RPW
    nv8 = d // 128
    nchunks = _RPW // _CH
    bufs = ((wq_a, wk_a, sem_a), (wq_b, wk_b, sem_b))

    def _zero(k, _):
        for u in range(8):
            qk_v[pl.ds(k * 128 + u * 16, 16)] = jnp.zeros((16,), jnp.float32)
        return 0

    lax.fori_loop(0, nv8, _zero, 0)
    pltpu.sync_copy(bq_hbm.at[pl.ds(base, _RPW)], bq_v.at[pl.ds(0, _RPW)])

    def _issue(c):
        wq_d, wk_d, sem = bufs[c % 2]
        row0 = base + c * _CH
        return (pltpu.async_copy(wq_hbm.at[pl.ds(row0, _CH)], wq_d, sem),
                pltpu.async_copy(wk_hbm.at[pl.ds(row0, _CH)], wk_d, sem))

    pending = _issue(0)
    for c in range(nchunks):
        for cp in pending:
            cp.wait()
        if c + 1 < nchunks:
            nxt = _issue(c + 1)
        wq_d, wk_d, _ = bufs[c % 2]
        bvec = bq_v[pl.ds(c * _CH, 16)]
        qrs = []
        for r in range(_CH):
            def _rowsum(k, acc, _r=r, _wq=wq_d):
                for u in range(8):
                    acc = acc + _wq[_r, pl.ds(k * 128 + u * 16, 16)]
                return acc

            acc = lax.fori_loop(0, nv8, _rowsum, jnp.zeros((16,), jnp.float32))
            qrs.append(_lane_allsum(acc) + bvec[r])

        def _accum(k, _, _wk=wk_d, _qrs=qrs):
            for u in range(8):
                sl = pl.ds(k * 128 + u * 16, 16)
                v = qk_v[sl]
                for r in range(_CH):
                    v = v + _qrs[r] * _wk[r, sl]
                qk_v[sl] = v
            return 0

        lax.fori_loop(0, nv8, _accum, 0)
        if c + 1 < nchunks:
            pending = nxt
    pltpu.sync_copy(qk_v, out_hbm.at[w])


def _qk_tc_kernel(wq_ref, bq_ref, wk_ref, qk_ref):
    i = pl.program_id(0)
    qc = jnp.sum(wq_ref[...], axis=1) + bq_ref[0, :]
    part = jnp.sum(qc[:, None] * wk_ref[...], axis=0, keepdims=True)

    @pl.when(i == 0)
    def _():
        qk_ref[...] = jnp.zeros_like(qk_ref)

    qk_ref[...] += part


def _tail_kernel(qk_in_ref, h_ref, wv_ref, bv_ref, we_ref, be_ref,
                 expert_ref, pmax_ref, logits_ref,
                 hbar_ref, acc_ref, m_ref, s_ref,
                 *, s2, s3, inv_scale):
    i = pl.program_id(0)

    @pl.when(i < s2)
    def _phase2():
        @pl.when(i == 0)
        def _():
            m_ref[...] = jnp.full_like(m_ref, -jnp.inf)
            s_ref[...] = jnp.zeros_like(s_ref)
            acc_ref[...] = jnp.zeros_like(acc_ref)

        h = h_ref[...]
        t = jax.lax.dot_general(
            h, qk_in_ref[0, :], (((2,), (0,)), ((), ())),
            preferred_element_type=jnp.float32, precision=_HI)
        t = t * inv_scale
        m_prev = m_ref[...]
        m_new = jnp.maximum(m_prev, jnp.max(t, axis=1, keepdims=True))
        alpha = jnp.exp(m_prev - m_new)
        p = jnp.exp(t - m_new)
        s_ref[...] = s_ref[...] * alpha + jnp.sum(p, axis=1, keepdims=True)
        pv = jax.lax.dot_general(
            p, h, (((1,), (1,)), ((0,), (0,))),
            preferred_element_type=jnp.float32)
        acc_ref[...] = acc_ref[...] * alpha + pv
        m_ref[...] = m_new

        @pl.when(i == s2 - 1)
        def _():
            hbar_ref[...] = acc_ref[...] / s_ref[...]

    @pl.when(i >= s2)
    def _phase3():
        hb = hbar_ref[...]
        hb_hi = hb.astype(jnp.bfloat16)
        hb_lo = (hb - hb_hi.astype(jnp.float32)).astype(jnp.bfloat16)
        hb2 = jnp.concatenate([hb_hi, hb_lo], axis=0)
        wv_hi = wv_ref[...].astype(jnp.bfloat16)
        bdim = hb.shape[0]
        rr = jax.lax.dot_general(
            hb2, wv_hi, (((1,), (1,)), ((), ())),
            preferred_element_type=jnp.float32)
        r = rr[:bdim, :] + rr[bdim:, :] + bv_ref[...]
        part = jax.lax.dot_general(
            r, we_ref[...], (((1,), (1,)), ((), ())),
            preferred_element_type=jnp.float32, precision=_HI)

        @pl.when(i == s2)
        def _():
            logits_ref[...] = jnp.zeros_like(logits_ref)

        logits_ref[...] += part

        @pl.when(i == s2 + s3 - 1)
        def _():
            logits = logits_ref[...] + be_ref[...]
            logits_ref[...] = logits
            row_max = jnp.max(logits, axis=1, keepdims=True)
            ex = jnp.exp(logits - row_max)
            denom = jnp.sum(ex, axis=1, keepdims=True)
            pmax_ref[...] = jnp.max(ex, axis=1, keepdims=True) / denom
            bd, ed = logits.shape
            idx = jax.lax.broadcasted_iota(jnp.int32, (bd, ed), 1)
            am = jnp.min(jnp.where(logits == row_max, idx, ed),
                         axis=1, keepdims=True)
            expert_ref[...] = (idx == am).astype(jnp.int32)


def kernel(h_dense, Wq, bq, Wk, bk, Wv, bv, We, be):
    del bk
    B, N, D = h_dense.shape
    E = We.shape[0]
    f32 = jnp.float32
    tc_rows = D - _SC_ROWS

    sc_fn = functools.partial(_sc_qk_kernel, d=D, base0=tc_rows)
    qk_sc_rows = pl.kernel(
        sc_fn,
        out_type=jax.ShapeDtypeStruct((_NW, D), f32),
        mesh=plsc.VectorSubcoreMesh(core_axis_name="c", subcore_axis_name="s"),
        scratch_types=[
            pltpu.VMEM((_CH, D), f32),
            pltpu.VMEM((_CH, D), f32),
            pltpu.VMEM((_CH, D), f32),
            pltpu.VMEM((_CH, D), f32),
            pltpu.VMEM((_RPW + 16,), f32),
            pltpu.VMEM((D,), f32),
            pltpu.SemaphoreType.DMA,
            pltpu.SemaphoreType.DMA,
        ],
    )(Wq, Wk, bq)

    C1 = 512
    s1 = tc_rows // C1
    qk_tc = pl.pallas_call(
        _qk_tc_kernel,
        grid=(s1,),
        in_specs=[
            pl.BlockSpec((C1, D), lambda i: (i, 0)),
            pl.BlockSpec((1, C1), lambda i: (0, i)),
            pl.BlockSpec((C1, D), lambda i: (i, 0)),
        ],
        out_specs=pl.BlockSpec((1, D), lambda i: (0, 0)),
        out_shape=jax.ShapeDtypeStruct((1, D), f32),
    )(Wq, bq.reshape(1, D), Wk)

    qk = qk_tc + jnp.sum(qk_sc_rows, axis=0, keepdims=True)

    C2, C3 = 256, 256
    s2, s3 = N // C2, D // C3
    l2, l3 = s2 - 1, s3 - 1

    def _clip(v, hi):
        return jnp.minimum(jnp.maximum(v, 0), hi)

    expert, pmax, logits = pl.pallas_call(
        functools.partial(_tail_kernel, s2=s2, s3=s3,
                          inv_scale=1.0 / (float(D) ** 0.5)),
        grid=(s2 + s3,),
        in_specs=[
            pl.BlockSpec((1, D), lambda i: (0, 0)),
            pl.BlockSpec((B, C2, D), lambda i: (0, _clip(i, l2), 0)),
            pl.BlockSpec((C3, D), lambda i: (_clip(i - s2, l3), 0)),
            pl.BlockSpec((1, C3), lambda i: (0, _clip(i - s2, l3))),
            pl.BlockSpec((E, C3), lambda i: (0, _clip(i - s2, l3))),
            pl.BlockSpec((1, E), lambda i: (0, 0)),
        ],
        out_specs=[
            pl.BlockSpec((B, E), lambda i: (0, 0)),
            pl.BlockSpec((B, 1), lambda i: (0, 0)),
            pl.BlockSpec((B, E), lambda i: (0, 0)),
        ],
        out_shape=[
            jax.ShapeDtypeStruct((B, E), jnp.int32),
            jax.ShapeDtypeStruct((B, 1), f32),
            jax.ShapeDtypeStruct((B, E), f32),
        ],
        scratch_shapes=[
            pltpu.VMEM((B, D), f32),
            pltpu.VMEM((B, D), f32),
            pltpu.VMEM((B, 1), f32),
            pltpu.VMEM((B, 1), f32),
        ],
    )(qk, h_dense, Wv, bv.reshape(1, D), We, be.reshape(1, E))

    return (expert, pmax, logits)


# speedup vs baseline: 1.7639x; 1.5183x over previous
import functools

import jax
import jax.numpy as jnp
from jax.experimental import pallas as pl
from jax.experimental.pallas import tpu as pltpu

_HI = jax.lax.Precision.HIGHEST


def _qk_kernel(wq_ref, bq_ref, wk_ref, qk_ref):
    i = pl.program_id(0)
    qc = jnp.sum(wq_ref[...], axis=1) + bq_ref[0, :]
    part = jnp.sum(qc[:, None] * wk_ref[...], axis=0, keepdims=True)

    @pl.when(i == 0)
    def _():
        qk_ref[...] = jnp.zeros_like(qk_ref)

    qk_ref[...] += part


def _flash_kernel(h_ref, qk_ref, hbar_ref, acc_ref, m_ref, s_ref, *, inv_scale):
    i = pl.program_id(0)
    nsteps = pl.num_programs(0)

    @pl.when(i == 0)
    def _():
        m_ref[...] = jnp.full_like(m_ref, -jnp.inf)
        s_ref[...] = jnp.zeros_like(s_ref)
        acc_ref[...] = jnp.zeros_like(acc_ref)

    h = h_ref[...]
    qk = qk_ref[0, :]
    t = jax.lax.dot_general(
        h, qk, (((2,), (0,)), ((), ())),
        preferred_element_type=jnp.float32, precision=_HI)
    t = t * inv_scale

    m_prev = m_ref[...]
    m_new = jnp.maximum(m_prev, jnp.max(t, axis=1, keepdims=True))
    alpha = jnp.exp(m_prev - m_new)
    p = jnp.exp(t - m_new)
    s_ref[...] = s_ref[...] * alpha + jnp.sum(p, axis=1, keepdims=True)
    pv = jax.lax.dot_general(
        p, h, (((1,), (1,)), ((0,), (0,))),
        preferred_element_type=jnp.float32)
    acc_ref[...] = acc_ref[...] * alpha + pv
    m_ref[...] = m_new

    @pl.when(i == nsteps - 1)
    def _():
        hbar_ref[...] = acc_ref[...] / s_ref[...]


def _tail_kernel(hbar_ref, wv_ref, bv_ref, we_ref, be_ref,
                 expert_ref, pmax_ref, logits_ref):
    i = pl.program_id(0)
    nsteps = pl.num_programs(0)
    hb = hbar_ref[...]
    hb_hi = hb.astype(jnp.bfloat16)
    hb_lo = (hb - hb_hi.astype(jnp.float32)).astype(jnp.bfloat16)
    hb2 = jnp.concatenate([hb_hi, hb_lo], axis=0)
    wv_hi = wv_ref[...].astype(jnp.bfloat16)
    bdim = hb.shape[0]
    rr = jax.lax.dot_general(
        hb2, wv_hi, (((1,), (1,)), ((), ())),
        preferred_element_type=jnp.float32)
    r = rr[:bdim, :] + rr[bdim:, :] + bv_ref[...]
    part = jax.lax.dot_general(
        r, we_ref[...], (((1,), (1,)), ((), ())),
        preferred_element_type=jnp.float32, precision=_HI)

    @pl.when(i == 0)
    def _():
        logits_ref[...] = jnp.zeros_like(logits_ref)

    logits_ref[...] += part

    @pl.when(i == nsteps - 1)
    def _():
        logits = logits_ref[...] + be_ref[...]
        logits_ref[...] = logits
        row_max = jnp.max(logits, axis=1, keepdims=True)
        ex = jnp.exp(logits - row_max)
        denom = jnp.sum(ex, axis=1, keepdims=True)
        pmax_ref[...] = jnp.max(ex, axis=1, keepdims=True) / denom
        bdim, edim = logits.shape
        idx = jax.lax.broadcasted_iota(jnp.int32, (bdim, edim), 1)
        am = jnp.min(jnp.where(logits == row_max, idx, edim),
                     axis=1, keepdims=True)
        expert_ref[...] = (idx == am).astype(jnp.int32)


def kernel(h_dense, Wq, bq, Wk, bk, Wv, bv, We, be):
    del bk
    B, N, D = h_dense.shape
    E = We.shape[0]
    f32 = jnp.float32

    C1 = 512
    qk = pl.pallas_call(
        _qk_kernel,
        grid=(D // C1,),
        in_specs=[
            pl.BlockSpec((C1, D), lambda i: (i, 0)),
            pl.BlockSpec((1, C1), lambda i: (0, i)),
            pl.BlockSpec((C1, D), lambda i: (i, 0)),
        ],
        out_specs=pl.BlockSpec((1, D), lambda i: (0, 0)),
        out_shape=jax.ShapeDtypeStruct((1, D), f32),
    )(Wq, bq.reshape(1, D), Wk)

    C2 = 256
    hbar = pl.pallas_call(
        functools.partial(_flash_kernel, inv_scale=1.0 / (float(D) ** 0.5)),
        grid=(N // C2,),
        in_specs=[
            pl.BlockSpec((B, C2, D), lambda i: (0, i, 0)),
            pl.BlockSpec((1, D), lambda i: (0, 0)),
        ],
        out_specs=pl.BlockSpec((B, D), lambda i: (0, 0)),
        out_shape=jax.ShapeDtypeStruct((B, D), f32),
        scratch_shapes=[
            pltpu.VMEM((B, D), f32),
            pltpu.VMEM((B, 1), f32),
            pltpu.VMEM((B, 1), f32),
        ],
    )(h_dense, qk)

    C3 = 512
    expert, pmax, logits = pl.pallas_call(
        _tail_kernel,
        grid=(D // C3,),
        in_specs=[
            pl.BlockSpec((B, D), lambda i: (0, 0)),
            pl.BlockSpec((C3, D), lambda i: (i, 0)),
            pl.BlockSpec((1, C3), lambda i: (0, i)),
            pl.BlockSpec((E, C3), lambda i: (0, i)),
            pl.BlockSpec((1, E), lambda i: (0, 0)),
        ],
        out_specs=[
            pl.BlockSpec((B, E), lambda i: (0, 0)),
            pl.BlockSpec((B, 1), lambda i: (0, 0)),
            pl.BlockSpec((B, E), lambda i: (0, 0)),
        ],
        out_shape=[
            jax.ShapeDtypeStruct((B, E), jnp.int32),
            jax.ShapeDtypeStruct((B, 1), f32),
            jax.ShapeDtypeStruct((B, E), f32),
        ],
    )(hbar, Wv, bv.reshape(1, D), We, be.reshape(1, E))

    return (expert, pmax, logits)
